# Initial kernel scaffold; baseline (speedup 1.0000x reference)
import jax
import jax.numpy as jnp
from jax.experimental import pallas as pl


def kernel(x, edge_index, W1, att_src1, att_dst1, b1, W2, att_src2, att_dst2, b2):
    raise NotImplementedError("write your pallas kernel here")


# speedup vs baseline: 33.2389x; 33.2389x over previous
import functools

import jax
import jax.numpy as jnp
from jax import lax
from jax.experimental import pallas as pl
from jax.experimental.pallas import tpu as pltpu
from jax.experimental.pallas import tpu_sc as plsc

NC, NS = 2, 16
K = 80
W1R = 144
W2R = 16


def _tc_a_body(x_ref, w1_ref, asv_ref, adv_ref, hcat_ref, asrc_ref, adst_ref):
    h = jnp.dot(x_ref[...], w1_ref[...], preferred_element_type=jnp.float32)
    bn = h.shape[0]
    hh = h.reshape(bn, 2, 128)
    asv = asv_ref[...]
    adv = adv_ref[...]
    asrc = jnp.sum(hh * asv[None], axis=-1)
    adst = jnp.sum(hh * adv[None], axis=-1)
    ht = jnp.transpose(hh, (1, 0, 2))
    pad = jnp.zeros((2, bn, W1R - 128), jnp.float32)
    hcat_ref[...] = jnp.concatenate([ht, pad], axis=-1)
    asrc_ref[...] = asrc
    adst_ref[...] = adst


def _tc_b_body(agg_ref, w2_ref, b1_ref, a2s_ref, a2d_ref, h2_ref, a2_ref):
    num0 = agg_ref[0, :, 0:128]
    den0 = agg_ref[0, :, 128:129]
    num1 = agg_ref[1, :, 0:128]
    den1 = agg_ref[1, :, 128:129]
    o = jnp.concatenate([num0 / (den0 + 1e-16), num1 / (den1 + 1e-16)], axis=-1)
    o = jax.nn.relu(o + b1_ref[...])
    h2 = jnp.dot(o, w2_ref[...], preferred_element_type=jnp.float32)
    h2_ref[...] = h2
    a2s = jnp.sum(h2 * a2s_ref[...], axis=-1)
    a2d = jnp.sum(h2 * a2d_ref[...], axis=-1)
    a2_ref[...] = jnp.stack([a2s, a2d], axis=1)


def _tc_c_body(agg_ref, b2_ref, out_ref):
    s = agg_ref[0] + agg_ref[1]
    o = s[:, 0:4] / (s[:, 4:5] + 1e-16) + b2_ref[...]
    m = jnp.max(o, axis=-1, keepdims=True)
    lse = m + jnp.log(jnp.sum(jnp.exp(o - m), axis=-1, keepdims=True))
    out_ref[...] = o - lse


def _make_sc1(n, e):
    ept = e // NS
    nchunks = ept // K
    rpt = n // NS
    zr = 25
    mesh = plsc.VectorSubcoreMesh(core_axis_name="c", subcore_axis_name="s",
                                  num_cores=NC, num_subcores=NS)

    @functools.partial(
        pl.kernel,
        out_type=jax.ShapeDtypeStruct((2, n, W1R), jnp.float32),
        mesh=mesh,
        compiler_params=pltpu.CompilerParams(use_tc_tiling_on_sc=False,
                                             needs_layout_passes=False),
        scratch_types=[
            pltpu.VMEM_SHARED((n, W1R), jnp.float32),
            pltpu.VMEM((n,), jnp.float32),
            pltpu.VMEM((n,), jnp.float32),
            pltpu.VMEM((K,), jnp.int32),
            pltpu.VMEM((K,), jnp.int32),
            pltpu.VMEM((K, W1R), jnp.float32),
            pltpu.VMEM((zr, W1R), jnp.float32),
            pltpu.SemaphoreType.DMA,
            pltpu.SemaphoreType.DMA,
        ],
    )
    def sc1(hcat_hbm, edges_hbm, asrc_hbm, adst_hbm, out_hbm,
            acc, asrc_t, adst_t, srcb, dstb, rows, zb,
            sem_g, sem_s):
        c = lax.axis_index("c")
        s = lax.axis_index("s")
        zv = jnp.zeros((16,), jnp.float32)
        for r in range(zr):
            for g in range(W1R // 16):
                zb[r, pl.ds(g * 16, 16)] = zv
        for t in range(rpt // zr):
            pltpu.sync_copy(zb, acc.at[pl.ds(s * rpt + t * zr, zr)])
        pltpu.sync_copy(asrc_hbm.at[c], asrc_t)
        pltpu.sync_copy(adst_hbm.at[c], adst_t)
        plsc.subcore_barrier()

        e_base = s * ept

        def chunk(k, carry):
            e0 = e_base + k * K
            pltpu.sync_copy(edges_hbm.at[0, pl.ds(e0, K)], srcb)
            pltpu.sync_copy(edges_hbm.at[1, pl.ds(e0, K)], dstb)
            pltpu.async_copy(hcat_hbm.at[c].at[srcb], rows, sem_g).wait()
            c128 = jnp.full((16,), 128, jnp.int32)
            ws = []
            for g in range(K // 16):
                d16 = pl.ds(g * 16, 16)
                a = (plsc.load_gather(asrc_t, [srcb[d16]])
                     + plsc.load_gather(adst_t, [dstb[d16]]))
                al = jnp.where(a >= 0, a, 0.2 * a)
                w = jnp.exp(al)
                ws.append(w)
                ri = lax.iota(jnp.int32, 16) + g * 16
                plsc.store_scatter(rows, [ri, c128], w)
            for i in range(K):
                wb = jnp.broadcast_to(ws[i // 16][i % 16], (16,))
                for g in range(8):
                    d16 = pl.ds(g * 16, 16)
                    rows[i, d16] = rows[i, d16] * wb
            pltpu.async_copy(rows, acc.at[dstb], sem_s, add=True).wait()
            return carry

        lax.fori_loop(0, nchunks, chunk, 0)
        plsc.subcore_barrier()
        pltpu.sync_copy(acc.at[pl.ds(s * rpt, rpt)],
                        out_hbm.at[c, pl.ds(s * rpt, rpt)])

    return sc1


def _make_sc2(n, e):
    ept = e // (NC * NS)
    nchunks = ept // K
    rpt = n // NS
    zr = 25
    mesh = plsc.VectorSubcoreMesh(core_axis_name="c", subcore_axis_name="s",
                                  num_cores=NC, num_subcores=NS)

    @functools.partial(
        pl.kernel,
        out_type=jax.ShapeDtypeStruct((2, n, W2R), jnp.float32),
        mesh=mesh,
        compiler_params=pltpu.CompilerParams(use_tc_tiling_on_sc=False,
                                             needs_layout_passes=False),
        scratch_types=[
            pltpu.VMEM_SHARED((n, W2R), jnp.float32),
            pltpu.VMEM((n, 4), jnp.float32),
            pltpu.VMEM((n,), jnp.float32),
            pltpu.VMEM((n,), jnp.float32),
            pltpu.VMEM((K,), jnp.int32),
            pltpu.VMEM((K,), jnp.int32),
            pltpu.VMEM((K, W2R), jnp.float32),
            pltpu.VMEM((zr, W2R), jnp.float32),
            pltpu.SemaphoreType.DMA,
        ],
    )
    def sc2(h2_hbm, edges_hbm, a2_hbm, out_hbm,
            acc, h2t, a2s_t, a2d_t, srcb, dstb, rows, zb, sem_s):
        c = lax.axis_index("c")
        s = lax.axis_index("s")
        zv = jnp.zeros((16,), jnp.float32)
        for r in range(zr):
            zb[r, :] = zv
        for t in range(rpt // zr):
            pltpu.sync_copy(zb, acc.at[pl.ds(s * rpt + t * zr, zr)])
        for r in range(K):
            rows[r, :] = zv
        pltpu.sync_copy(h2_hbm, h2t)
        pltpu.sync_copy(a2_hbm.at[0], a2s_t)
        pltpu.sync_copy(a2_hbm.at[1], a2d_t)
        plsc.subcore_barrier()

        wid = c * NS + s
        e_base = wid * ept

        def chunk(k, carry):
            e0 = e_base + k * K
            pltpu.sync_copy(edges_hbm.at[0, pl.ds(e0, K)], srcb)
            pltpu.sync_copy(edges_hbm.at[1, pl.ds(e0, K)], dstb)
            for g in range(K // 16):
                d16 = pl.ds(g * 16, 16)
                sv = srcb[d16]
                dv = dstb[d16]
                a = (plsc.load_gather(a2s_t, [sv])
                     + plsc.load_gather(a2d_t, [dv]))
                al = jnp.where(a >= 0, a, 0.2 * a)
                w = jnp.exp(al)
                ri = lax.iota(jnp.int32, 16) + g * 16
                for j in range(4):
                    cj = jnp.full((16,), j, jnp.int32)
                    hv = plsc.load_gather(h2t, [sv, cj])
                    plsc.store_scatter(rows, [ri, cj], hv * w)
                plsc.store_scatter(rows, [ri, jnp.full((16,), 4, jnp.int32)], w)
            pltpu.async_copy(rows, acc.at[dstb], sem_s, add=True).wait()
            return carry

        lax.fori_loop(0, nchunks, chunk, 0)
        plsc.subcore_barrier()
        pltpu.sync_copy(acc.at[pl.ds(s * rpt, rpt)],
                        out_hbm.at[c, pl.ds(s * rpt, rpt)])

    return sc2


def kernel(x, edge_index, W1, att_src1, att_dst1, b1, W2, att_src2, att_dst2, b2):
    n, f_in = x.shape
    e = edge_index.shape[1]
    hid = W1.shape[1] // 2
    out_ch = W2.shape[1]
    bn = 1000

    asv = att_src1.reshape(2, hid)
    adv = att_dst1.reshape(2, hid)

    tc_a = pl.pallas_call(
        _tc_a_body,
        grid=(n // bn,),
        in_specs=[
            pl.BlockSpec((bn, f_in), lambda i: (i, 0)),
            pl.BlockSpec((f_in, 2 * hid), lambda i: (0, 0)),
            pl.BlockSpec((2, hid), lambda i: (0, 0)),
            pl.BlockSpec((2, hid), lambda i: (0, 0)),
        ],
        out_specs=[
            pl.BlockSpec((2, bn, W1R), lambda i: (0, i, 0)),
            pl.BlockSpec((bn, 2), lambda i: (i, 0)),
            pl.BlockSpec((bn, 2), lambda i: (i, 0)),
        ],
        out_shape=[
            jax.ShapeDtypeStruct((2, n, W1R), jnp.float32),
            jax.ShapeDtypeStruct((n, 2), jnp.float32),
            jax.ShapeDtypeStruct((n, 2), jnp.float32),
        ],
    )
    hcat3, asrc, adst = tc_a(x, W1, asv, adv)
    asrc = asrc.T
    adst = adst.T

    sc1 = _make_sc1(n, e)
    agg1 = sc1(hcat3, edge_index, asrc, adst)

    a2sv = att_src2.reshape(1, out_ch)
    a2dv = att_dst2.reshape(1, out_ch)
    tc_b = pl.pallas_call(
        _tc_b_body,
        grid=(n // bn,),
        in_specs=[
            pl.BlockSpec((2, bn, W1R), lambda i: (0, i, 0)),
            pl.BlockSpec((2 * hid, out_ch), lambda i: (0, 0)),
            pl.BlockSpec((1, 2 * hid), lambda i: (0, 0)),
            pl.BlockSpec((1, out_ch), lambda i: (0, 0)),
            pl.BlockSpec((1, out_ch), lambda i: (0, 0)),
        ],
        out_specs=[
            pl.BlockSpec((bn, out_ch), lambda i: (i, 0)),
            pl.BlockSpec((bn, 2), lambda i: (i, 0)),
        ],
        out_shape=[
            jax.ShapeDtypeStruct((n, out_ch), jnp.float32),
            jax.ShapeDtypeStruct((n, 2), jnp.float32),
        ],
    )
    h2, a2 = tc_b(agg1, W2, b1.reshape(1, -1), a2sv, a2dv)
    a2 = a2.T

    sc2 = _make_sc2(n, e)
    agg2 = sc2(h2, edge_index, a2)

    tc_c = pl.pallas_call(
        _tc_c_body,
        grid=(n // bn,),
        in_specs=[
            pl.BlockSpec((2, bn, W2R), lambda i: (0, i, 0)),
            pl.BlockSpec((1, out_ch), lambda i: (0, 0)),
        ],
        out_specs=pl.BlockSpec((bn, out_ch), lambda i: (i, 0)),
        out_shape=jax.ShapeDtypeStruct((n, out_ch), jnp.float32),
    )
    return tc_c(agg2, b2.reshape(1, -1))


# speedup vs baseline: 43.7960x; 1.3176x over previous
import functools

import jax
import jax.numpy as jnp
from jax import lax
from jax.experimental import pallas as pl
from jax.experimental.pallas import tpu as pltpu
from jax.experimental.pallas import tpu_sc as plsc

NC, NS = 2, 16
K = 80
W1R = 136
W2R = 16


def _tc_a_body(x_ref, w1_ref, asv_ref, adv_ref, hcat_ref, asrc_ref, adst_ref):
    h = jnp.dot(x_ref[...], w1_ref[...], preferred_element_type=jnp.float32)
    bn = h.shape[0]
    hh = h.reshape(bn, 2, 128)
    asv = asv_ref[...]
    adv = adv_ref[...]
    asrc = jnp.sum(hh * asv[None], axis=-1)
    adst = jnp.sum(hh * adv[None], axis=-1)
    ht = jnp.transpose(hh, (1, 0, 2))
    pad = jnp.zeros((2, bn, W1R - 128), jnp.float32)
    hcat_ref[...] = jnp.concatenate([ht, pad], axis=-1)
    asrc_ref[...] = asrc
    adst_ref[...] = adst


def _tc_b_body(agg_ref, w2_ref, b1_ref, a2s_ref, a2d_ref, h2_ref, a2_ref):
    num0 = agg_ref[0, :, 0:128]
    den0 = agg_ref[0, :, 128:129]
    num1 = agg_ref[1, :, 0:128]
    den1 = agg_ref[1, :, 128:129]
    o = jnp.concatenate([num0 / (den0 + 1e-16), num1 / (den1 + 1e-16)], axis=-1)
    o = jax.nn.relu(o + b1_ref[...])
    h2 = jnp.dot(o, w2_ref[...], preferred_element_type=jnp.float32)
    h2_ref[...] = h2
    a2s = jnp.sum(h2 * a2s_ref[...], axis=-1)
    a2d = jnp.sum(h2 * a2d_ref[...], axis=-1)
    a2_ref[...] = jnp.stack([a2s, a2d], axis=1)


def _tc_c_body(agg_ref, b2_ref, out_ref):
    s = agg_ref[0] + agg_ref[1]
    o = s[:, 0:4] / (s[:, 4:5] + 1e-16) + b2_ref[...]
    m = jnp.max(o, axis=-1, keepdims=True)
    lse = m + jnp.log(jnp.sum(jnp.exp(o - m), axis=-1, keepdims=True))
    out_ref[...] = o - lse


def _make_sc1(n, e):
    ept = e // NS
    nchunks = ept // K
    rpt = n // NS
    mesh = plsc.VectorSubcoreMesh(core_axis_name="c", subcore_axis_name="s",
                                  num_cores=NC, num_subcores=NS)

    @functools.partial(
        pl.kernel,
        out_type=jax.ShapeDtypeStruct((2, n, W1R), jnp.float32),
        mesh=mesh,
        compiler_params=pltpu.CompilerParams(use_tc_tiling_on_sc=False,
                                             needs_layout_passes=False),
        scratch_types=[
            pltpu.VMEM_SHARED((n, W1R), jnp.float32),
            pltpu.VMEM((n,), jnp.float32),
            pltpu.VMEM((n,), jnp.float32),
            pltpu.VMEM((K,), jnp.int32),
            pltpu.VMEM((K,), jnp.int32),
            pltpu.VMEM((K,), jnp.int32),
            pltpu.VMEM((K,), jnp.int32),
            pltpu.VMEM((K, W1R), jnp.float32),
            pltpu.VMEM((K, W1R), jnp.float32),
            pltpu.SemaphoreType.DMA,
            pltpu.SemaphoreType.DMA,
            pltpu.SemaphoreType.DMA,
        ],
    )
    def sc1(hcat_hbm, edges_hbm, asrc_hbm, adst_hbm, zero_hbm, out_hbm,
            acc, asrc_t, adst_t, srcb0, srcb1, dstb0, dstb1, rows0, rows1,
            sem_g0, sem_g1, sem_s):
        srcbs, dstbs = (srcb0, srcb1), (dstb0, dstb1)
        rowss, sem_gs = (rows0, rows1), (sem_g0, sem_g1)
        c = lax.axis_index("c")
        s = lax.axis_index("s")
        pltpu.sync_copy(zero_hbm, acc.at[pl.ds(s * rpt, rpt)])
        pltpu.sync_copy(asrc_hbm.at[c], asrc_t)
        pltpu.sync_copy(adst_hbm.at[c], adst_t)
        plsc.subcore_barrier()

        e_base = s * ept
        c128 = jnp.full((16,), 128, jnp.int32)

        pltpu.sync_copy(edges_hbm.at[0, pl.ds(e_base, K)], srcbs[0])
        pltpu.sync_copy(edges_hbm.at[1, pl.ds(e_base, K)], dstbs[0])
        pltpu.async_copy(hcat_hbm.at[c].at[srcbs[0]], rowss[0], sem_gs[0])

        def pair(k2, carry):
            for b in (0, 1):
                nb = 1 - b
                pf = jnp.minimum(2 * k2 + b + 1, nchunks - 1)
                e0n = e_base + pf * K
                pltpu.sync_copy(edges_hbm.at[0, pl.ds(e0n, K)], srcbs[nb])
                pltpu.sync_copy(edges_hbm.at[1, pl.ds(e0n, K)], dstbs[nb])
                pltpu.async_copy(hcat_hbm.at[c].at[srcbs[nb]], rowss[nb],
                                 sem_gs[nb])
                pltpu.make_async_copy(hcat_hbm.at[c].at[srcbs[b]], rowss[b],
                                      sem_gs[b]).wait()
                ws = []
                for g in range(K // 16):
                    d16 = pl.ds(g * 16, 16)
                    a = (plsc.load_gather(asrc_t, [srcbs[b][d16]])
                         + plsc.load_gather(adst_t, [dstbs[b][d16]]))
                    al = jnp.where(a >= 0, a, 0.2 * a)
                    w = jnp.exp(al)
                    ws.append(w)
                    ri = lax.iota(jnp.int32, 16) + g * 16
                    plsc.store_scatter(rowss[b], [ri, c128], w)
                for i in range(K):
                    wb = jnp.broadcast_to(ws[i // 16][i % 16], (16,))
                    for g in range(8):
                        d16 = pl.ds(g * 16, 16)
                        rowss[b][i, d16] = rowss[b][i, d16] * wb
                pltpu.async_copy(rowss[b], acc.at[dstbs[b]], sem_s,
                                 add=True).wait()
            return carry

        lax.fori_loop(0, nchunks // 2, pair, 0)
        pltpu.make_async_copy(hcat_hbm.at[c].at[srcbs[0]], rowss[0],
                              sem_gs[0]).wait()
        plsc.subcore_barrier()
        pltpu.sync_copy(acc.at[pl.ds(s * rpt, rpt)],
                        out_hbm.at[c, pl.ds(s * rpt, rpt)])

    return sc1


def _make_sc2(n, e):
    ept = e // (NC * NS)
    nchunks = ept // K
    rpt = n // NS
    zr = 25
    mesh = plsc.VectorSubcoreMesh(core_axis_name="c", subcore_axis_name="s",
                                  num_cores=NC, num_subcores=NS)

    @functools.partial(
        pl.kernel,
        out_type=jax.ShapeDtypeStruct((2, n, W2R), jnp.float32),
        mesh=mesh,
        compiler_params=pltpu.CompilerParams(use_tc_tiling_on_sc=False,
                                             needs_layout_passes=False),
        scratch_types=[
            pltpu.VMEM_SHARED((n, W2R), jnp.float32),
            pltpu.VMEM((n, 4), jnp.float32),
            pltpu.VMEM((n,), jnp.float32),
            pltpu.VMEM((n,), jnp.float32),
            pltpu.VMEM((K,), jnp.int32),
            pltpu.VMEM((K,), jnp.int32),
            pltpu.VMEM((K, W2R), jnp.float32),
            pltpu.VMEM((zr, W2R), jnp.float32),
            pltpu.SemaphoreType.DMA,
        ],
    )
    def sc2(h2_hbm, edges_hbm, a2_hbm, out_hbm,
            acc, h2t, a2s_t, a2d_t, srcb, dstb, rows, zb, sem_s):
        c = lax.axis_index("c")
        s = lax.axis_index("s")
        zv = jnp.zeros((16,), jnp.float32)
        for r in range(zr):
            zb[r, :] = zv
        for t in range(rpt // zr):
            pltpu.sync_copy(zb, acc.at[pl.ds(s * rpt + t * zr, zr)])
        for r in range(K):
            rows[r, :] = zv
        pltpu.sync_copy(h2_hbm, h2t)
        pltpu.sync_copy(a2_hbm.at[0], a2s_t)
        pltpu.sync_copy(a2_hbm.at[1], a2d_t)
        plsc.subcore_barrier()

        wid = c * NS + s
        e_base = wid * ept

        def chunk(k, carry):
            e0 = e_base + k * K
            pltpu.sync_copy(edges_hbm.at[0, pl.ds(e0, K)], srcb)
            pltpu.sync_copy(edges_hbm.at[1, pl.ds(e0, K)], dstb)
            for g in range(K // 16):
                d16 = pl.ds(g * 16, 16)
                sv = srcb[d16]
                dv = dstb[d16]
                a = (plsc.load_gather(a2s_t, [sv])
                     + plsc.load_gather(a2d_t, [dv]))
                al = jnp.where(a >= 0, a, 0.2 * a)
                w = jnp.exp(al)
                ri = lax.iota(jnp.int32, 16) + g * 16
                for j in range(4):
                    cj = jnp.full((16,), j, jnp.int32)
                    hv = plsc.load_gather(h2t, [sv, cj])
                    plsc.store_scatter(rows, [ri, cj], hv * w)
                plsc.store_scatter(rows, [ri, jnp.full((16,), 4, jnp.int32)], w)
            pltpu.async_copy(rows, acc.at[dstb], sem_s, add=True).wait()
            return carry

        lax.fori_loop(0, nchunks, chunk, 0)
        plsc.subcore_barrier()
        pltpu.sync_copy(acc.at[pl.ds(s * rpt, rpt)],
                        out_hbm.at[c, pl.ds(s * rpt, rpt)])

    return sc2


def kernel(x, edge_index, W1, att_src1, att_dst1, b1, W2, att_src2, att_dst2, b2):
    n, f_in = x.shape
    e = edge_index.shape[1]
    hid = W1.shape[1] // 2
    out_ch = W2.shape[1]
    bn = 1000

    asv = att_src1.reshape(2, hid)
    adv = att_dst1.reshape(2, hid)

    tc_a = pl.pallas_call(
        _tc_a_body,
        grid=(n // bn,),
        in_specs=[
            pl.BlockSpec((bn, f_in), lambda i: (i, 0)),
            pl.BlockSpec((f_in, 2 * hid), lambda i: (0, 0)),
            pl.BlockSpec((2, hid), lambda i: (0, 0)),
            pl.BlockSpec((2, hid), lambda i: (0, 0)),
        ],
        out_specs=[
            pl.BlockSpec((2, bn, W1R), lambda i: (0, i, 0)),
            pl.BlockSpec((bn, 2), lambda i: (i, 0)),
            pl.BlockSpec((bn, 2), lambda i: (i, 0)),
        ],
        out_shape=[
            jax.ShapeDtypeStruct((2, n, W1R), jnp.float32),
            jax.ShapeDtypeStruct((n, 2), jnp.float32),
            jax.ShapeDtypeStruct((n, 2), jnp.float32),
        ],
    )
    hcat3, asrc, adst = tc_a(x, W1, asv, adv)
    asrc = asrc.T
    adst = adst.T

    sc1 = _make_sc1(n, e)
    zfeed = jnp.zeros((n // NS, W1R), jnp.float32)
    agg1 = sc1(hcat3, edge_index, asrc, adst, zfeed)

    a2sv = att_src2.reshape(1, out_ch)
    a2dv = att_dst2.reshape(1, out_ch)
    tc_b = pl.pallas_call(
        _tc_b_body,
        grid=(n // bn,),
        in_specs=[
            pl.BlockSpec((2, bn, W1R), lambda i: (0, i, 0)),
            pl.BlockSpec((2 * hid, out_ch), lambda i: (0, 0)),
            pl.BlockSpec((1, 2 * hid), lambda i: (0, 0)),
            pl.BlockSpec((1, out_ch), lambda i: (0, 0)),
            pl.BlockSpec((1, out_ch), lambda i: (0, 0)),
        ],
        out_specs=[
            pl.BlockSpec((bn, out_ch), lambda i: (i, 0)),
            pl.BlockSpec((bn, 2), lambda i: (i, 0)),
        ],
        out_shape=[
            jax.ShapeDtypeStruct((n, out_ch), jnp.float32),
            jax.ShapeDtypeStruct((n, 2), jnp.float32),
        ],
    )
    h2, a2 = tc_b(agg1, W2, b1.reshape(1, -1), a2sv, a2dv)
    a2 = a2.T

    sc2 = _make_sc2(n, e)
    agg2 = sc2(h2, edge_index, a2)

    tc_c = pl.pallas_call(
        _tc_c_body,
        grid=(n // bn,),
        in_specs=[
            pl.BlockSpec((2, bn, W2R), lambda i: (0, i, 0)),
            pl.BlockSpec((1, out_ch), lambda i: (0, 0)),
        ],
        out_specs=pl.BlockSpec((bn, out_ch), lambda i: (i, 0)),
        out_shape=jax.ShapeDtypeStruct((n, out_ch), jnp.float32),
    )
    return tc_c(agg2, b2.reshape(1, -1))
